# baseline (speedup 1.0000x reference)
import functools

import jax
import jax.numpy as jnp
from jax import lax
from jax.experimental import pallas as pl
from jax.experimental.pallas import tpu as pltpu
from jax.experimental.pallas import tpu_sc as plsc

EMBED_DIM = 32
BATCH = 16384
N_ROWS = 1000000

NC = 2
NS = 16
L = 16
NW = NC * NS
B_PER_W = BATCH // NW
N_CHUNKS = B_PER_W // L


def _gmf_body(users_hbm, items_hbm, utab_hbm, itab_hbm, w_hbm, b_hbm,
              out_hbm, idx_u, idx_i, fidx_u, fidx_i, rows_u, rows_i,
              wv, bv, out_v, sem_u, sem_i):
    wid = lax.axis_index("s") * NC + lax.axis_index("c")
    base = wid * B_PER_W

    pltpu.sync_copy(users_hbm.at[pl.ds(base, B_PER_W)], idx_u)
    pltpu.sync_copy(items_hbm.at[pl.ds(base, B_PER_W)], idx_i)
    pltpu.sync_copy(w_hbm, wv)
    pltpu.sync_copy(b_hbm, bv.at[pl.ds(0, 1)])

    def fidx_body(i, _):
        off = i * L
        u16 = idx_u[pl.ds(off, L)]
        i16 = idx_i[pl.ds(off, L)]
        for d in range(EMBED_DIM):
            fidx_u[d, pl.ds(off, L)] = u16 + (d * N_ROWS)
            fidx_i[d, pl.ds(off, L)] = i16 + (d * N_ROWS)
        return ()
    lax.fori_loop(0, B_PER_W // L, fidx_body, (), unroll=False)

    cps = []
    for d in range(EMBED_DIM):
        cps.append(pltpu.async_copy(
            utab_hbm.at[fidx_u.at[d]], rows_u.at[d], sem_u))
        cps.append(pltpu.async_copy(
            itab_hbm.at[fidx_i.at[d]], rows_i.at[d], sem_i))
    for cp in cps:
        cp.wait()

    b_s = bv[pl.ds(0, L)][0]
    w_lo = wv[pl.ds(0, L)]
    w_hi = wv[pl.ds(L, L)]
    w_s = [w_lo[d] for d in range(L)] + [w_hi[d] for d in range(L)]

    def chunk_body(c, _):
        off = c * L
        acc = jnp.full((L,), b_s, dtype=jnp.float32)
        for d in range(EMBED_DIM):
            u_g = rows_u[d, pl.ds(off, L)]
            i_g = rows_i[d, pl.ds(off, L)]
            acc = acc + (u_g * i_g) * w_s[d]
        out_v[pl.ds(off, L)] = acc
        return ()

    lax.fori_loop(0, N_CHUNKS, chunk_body, (), unroll=False)

    pltpu.sync_copy(out_v, out_hbm.at[pl.ds(base, B_PER_W)])


@jax.jit
def _gmf(users, items, ut_flat, it_flat, w, b):
    mesh = plsc.VectorSubcoreMesh(core_axis_name="c", subcore_axis_name="s")
    run = functools.partial(
        pl.kernel,
        mesh=mesh,
        out_type=jax.ShapeDtypeStruct((BATCH,), jnp.float32),
        compiler_params=pltpu.CompilerParams(
            needs_layout_passes=False, use_tc_tiling_on_sc=False),
        scratch_types=[
            pltpu.VMEM((B_PER_W,), jnp.int32),
            pltpu.VMEM((B_PER_W,), jnp.int32),
            pltpu.VMEM((EMBED_DIM, B_PER_W), jnp.int32),
            pltpu.VMEM((EMBED_DIM, B_PER_W), jnp.int32),
            pltpu.VMEM((EMBED_DIM, B_PER_W), jnp.float32),
            pltpu.VMEM((EMBED_DIM, B_PER_W), jnp.float32),
            pltpu.VMEM((EMBED_DIM,), jnp.float32),
            pltpu.VMEM((L,), jnp.float32),
            pltpu.VMEM((B_PER_W,), jnp.float32),
            pltpu.SemaphoreType.DMA,
            pltpu.SemaphoreType.DMA,
        ],
    )(_gmf_body)
    return run(users, items, ut_flat, it_flat, w, b)


def kernel(users, items, user_table, item_table, W, b):
    w_flat = W.reshape(EMBED_DIM)
    ut_flat = user_table.T.reshape(-1)
    it_flat = item_table.T.reshape(-1)
    return _gmf(users, items, ut_flat, it_flat, w_flat, b)
